# baseline (speedup 1.0000x reference)
import functools

import jax
import jax.numpy as jnp
from jax import lax
from jax.experimental import pallas as pl
from jax.experimental.pallas import tpu as pltpu
from jax.experimental.pallas import tpu_sc as plsc

N = 10000
E = 320000
HID = 256
NC = 2
NS = 16
NW = NC * NS
NPAD = 10240
NPAD2 = 2 * NPAD
SLICE = NPAD // NS
EPT = E // NW
BLK = 2000
NBLK = EPT // BLK
UN = 5
OUTPAD = 128

_mesh = plsc.VectorSubcoreMesh(core_axis_name="c", subcore_axis_name="s")
_sc_params = pltpu.CompilerParams(needs_layout_passes=False)


def _merge(dst_ref, src_ref):
    def body(i, _):
        for u in range(UN):
            sl = pl.ds((i * UN + u) * 16, 16)
            dst_ref[sl] = dst_ref[sl] + src_ref[sl]
        return 0

    lax.fori_loop(0, NPAD // 16 // UN, body, 0)


def _rne_bf16_hi(u):
    lsb = lax.shift_right_logical(u, 16) & 1
    return (u + 0x7FFF + lsb) & jnp.int32(-65536)


@functools.partial(
    pl.kernel,
    out_type=(
        jax.ShapeDtypeStruct((NW * NPAD,), jnp.float32),
        jax.ShapeDtypeStruct((NPAD,), jnp.int32),
    ),
    mesh=_mesh,
    compiler_params=_sc_params,
    scratch_types=[
        pltpu.VMEM((BLK,), jnp.int32),
        pltpu.VMEM((BLK,), jnp.int32),
        pltpu.VMEM((NPAD,), jnp.float32),
        pltpu.VMEM((NPAD,), jnp.float32),
        pltpu.VMEM((SLICE, 2), jnp.float32),
        pltpu.VMEM((SLICE,), jnp.int32),
        pltpu.SemaphoreType.DMA,
        pltpu.SemaphoreType.DMA,
    ],
)
def _deg_kernel(edge_hbm, x_hbm, zeros_hbm, deg_out, xpk_out,
                idx0, idx1, acc0, acc1, xsl, xpk, sem0, sem1):
    cid = lax.axis_index("c")
    sid = lax.axis_index("s")
    wid = cid * NS + sid
    base = wid * EPT
    bufs = (idx0, idx1)
    sems = (sem0, sem1)
    ones16 = jnp.ones((16,), jnp.float32)
    iota16 = jax.lax.iota(jnp.int32, 16)

    d0 = pltpu.async_copy(edge_hbm.at[pl.ds(E + base, BLK)], idx0, sem0)
    pltpu.sync_copy(zeros_hbm, acc0)
    pltpu.sync_copy(zeros_hbm, acc1)
    descs = [d0]
    accs = (acc0, acc1)

    @pl.when(cid == 0)
    def _pack():
        nbase = sid * SLICE

        @pl.when(nbase + SLICE <= N)
        def _copy_full():
            pltpu.sync_copy(x_hbm.at[pl.ds(nbase, SLICE), :], xsl)

        @pl.when(nbase + SLICE > N)
        def _copy_tail():
            pltpu.sync_copy(x_hbm.at[pl.ds(N - 400, 400), :],
                            xsl.at[pl.ds(0, 400), :])

        def pbody(i, _):
            off = i * 16
            rows = off + iota16
            ga = plsc.load_gather(xsl, [rows, jnp.zeros((16,), jnp.int32)])
            gb = plsc.load_gather(xsl, [rows, jnp.ones((16,), jnp.int32)])
            valid = (nbase + rows) < N
            ua = _rne_bf16_hi(plsc.bitcast(ga, jnp.int32))
            ub = _rne_bf16_hi(plsc.bitcast(gb, jnp.int32))
            pk = ua | lax.shift_right_logical(ub, 16)
            xpk[pl.ds(off, 16)] = jnp.where(valid, pk, 0)
            return 0

        lax.fori_loop(0, SLICE // 16, pbody, 0)
        pltpu.sync_copy(xpk, xpk_out.at[pl.ds(sid * SLICE, SLICE)])

    for j in range(NBLK):
        if j + 1 < NBLK:
            descs.append(pltpu.async_copy(
                edge_hbm.at[pl.ds(E + base + (j + 1) * BLK, BLK)],
                bufs[(j + 1) % 2], sems[(j + 1) % 2]))
        descs[j].wait()
        buf = bufs[j % 2]

        def inner(i, _):
            for u in range(UN):
                d = buf[pl.ds((i * UN + u) * 16, 16)]
                plsc.addupdate_scatter(accs[u % 2], [d], ones16)
            return 0

        lax.fori_loop(0, BLK // 16 // UN, inner, 0)

    _merge(acc0, acc1)
    pltpu.sync_copy(acc0, deg_out.at[pl.ds(wid * NPAD, NPAD)])


def _scale_body(deg_parts, dinv_o):
    deg = jnp.sum(deg_parts[...].reshape(NW, NPAD // 128, 128), axis=0) + 1.0
    dinv_o[...] = 1.0 / jnp.sqrt(deg)


_scale_kernel = pl.pallas_call(
    _scale_body,
    out_shape=jax.ShapeDtypeStruct((NPAD // 128, 128), jnp.float32),
)


@functools.partial(
    pl.kernel,
    out_type=(
        jax.ShapeDtypeStruct((NW * NPAD,), jnp.float32),
        jax.ShapeDtypeStruct((NW * NPAD,), jnp.float32),
    ),
    mesh=_mesh,
    compiler_params=_sc_params,
    scratch_types=[
        pltpu.VMEM((NPAD,), jnp.int32),
        pltpu.VMEM((NPAD,), jnp.float32),
        pltpu.VMEM((NPAD,), jnp.float32),
        pltpu.VMEM((NPAD,), jnp.float32),
        pltpu.VMEM((NPAD,), jnp.float32),
        pltpu.VMEM((NPAD,), jnp.float32),
        pltpu.VMEM((BLK,), jnp.int32),
        pltpu.VMEM((BLK,), jnp.int32),
        pltpu.VMEM((BLK,), jnp.int32),
        pltpu.VMEM((BLK,), jnp.int32),
        pltpu.SemaphoreType.DMA,
        pltpu.SemaphoreType.DMA,
    ],
)
def _agg_kernel(edge_hbm, xpk_hbm, dinv_hbm, zeros_hbm, a_out, b_out,
                xpk_v, dinv_v, acc_a0, acc_b0, acc_a1, acc_b1,
                src0, dst0, src1, dst1, sem0, sem1):
    cid = lax.axis_index("c")
    sid = lax.axis_index("s")
    wid = cid * NS + sid
    base = wid * EPT
    sbufs = (src0, src1)
    dbufs = (dst0, dst1)
    sems = (sem0, sem1)
    accs = ((acc_a0, acc_b0), (acc_a1, acc_b1))

    descs = [(pltpu.async_copy(edge_hbm.at[pl.ds(base, BLK)], src0, sem0),
              pltpu.async_copy(edge_hbm.at[pl.ds(E + base, BLK)], dst0, sem0))]
    pltpu.sync_copy(xpk_hbm, xpk_v)
    pltpu.sync_copy(dinv_hbm, dinv_v)
    pltpu.sync_copy(zeros_hbm, acc_a0)
    pltpu.sync_copy(zeros_hbm, acc_b0)
    pltpu.sync_copy(zeros_hbm, acc_a1)
    pltpu.sync_copy(zeros_hbm, acc_b1)

    def _unpack(pk):
        va = plsc.bitcast(pk & jnp.int32(-65536), jnp.float32)
        vb = plsc.bitcast(lax.shift_left(pk, 16), jnp.float32)
        return va, vb

    @pl.when(cid == 0)
    def _selfloop():
        nbase = sid * SLICE

        def sbody(i, _):
            off = nbase + i * 16
            va, vb = _unpack(xpk_v[pl.ds(off, 16)])
            dv = dinv_v[pl.ds(off, 16)]
            acc_a0[pl.ds(off, 16)] = va * dv
            acc_b0[pl.ds(off, 16)] = vb * dv
            return 0

        lax.fori_loop(0, SLICE // 16, sbody, 0)

    for j in range(NBLK):
        if j + 1 < NBLK:
            nb = (j + 1) % 2
            descs.append((
                pltpu.async_copy(
                    edge_hbm.at[pl.ds(base + (j + 1) * BLK, BLK)],
                    sbufs[nb], sems[nb]),
                pltpu.async_copy(
                    edge_hbm.at[pl.ds(E + base + (j + 1) * BLK, BLK)],
                    dbufs[nb], sems[nb])))
        descs[j][0].wait()
        descs[j][1].wait()
        sbuf = sbufs[j % 2]
        dbuf = dbufs[j % 2]

        def inner(i, _):
            for u in range(UN):
                sl = pl.ds((i * UN + u) * 16, 16)
                s = sbuf[sl]
                d = dbuf[sl]
                va, vb = _unpack(plsc.load_gather(xpk_v, [s]))
                dv = plsc.load_gather(dinv_v, [s])
                aa, bb = accs[u % 2]
                plsc.addupdate_scatter(aa, [d], va * dv)
                plsc.addupdate_scatter(bb, [d], vb * dv)
            return 0

        lax.fori_loop(0, BLK // 16 // UN, inner, 0)

    _merge(acc_a0, acc_a1)
    _merge(acc_b0, acc_b1)
    pltpu.sync_copy(acc_a0, a_out.at[pl.ds(wid * NPAD, NPAD)])
    pltpu.sync_copy(acc_b0, b_out.at[pl.ds(wid * NPAD, NPAD)])


def _final_body(acca, accb, dinv, gw, gb, lnw, lnb, hw, hb, ow, ob, out_ref):
    CH = 1024
    R = NPAD // 128

    A = acca[pl.ds(0, R), :]
    B = accb[pl.ds(0, R), :]
    for w in range(1, NW):
        A = A + acca[pl.ds(w * R, R), :]
        B = B + accb[pl.ds(w * R, R), :]
    DV = dinv[...]

    def _bf16r(v):
        return v.astype(jnp.bfloat16).astype(jnp.float32)

    gwr = _bf16r(gw[...])
    pooled = jnp.zeros((1, HID), jnp.float32)
    for i in range(NPAD // CH):
        aa = A[i * 8:(i + 1) * 8, :].reshape(CH)
        bb = B[i * 8:(i + 1) * 8, :].reshape(CH)
        dv = DV[i * 8:(i + 1) * 8, :].reshape(CH)
        g = ((aa * dv)[:, None] * gwr[0][None, :]
             + (bb * dv)[:, None] * gwr[1][None, :] + gb[...])
        g = jnp.maximum(g, 0.0)
        mean = jnp.mean(g, axis=1, keepdims=True)
        cen = g - mean
        var = jnp.mean(cen * cen, axis=1, keepdims=True)
        xln = cen / jnp.sqrt(var + 1e-5) * lnw[...] + lnb[...]
        rows = i * CH + lax.broadcasted_iota(jnp.int32, (CH, 1), 0)
        xln = jnp.where(rows < N, xln, 0.0)
        pooled = pooled + jnp.sum(xln, axis=0, keepdims=True)

    h = jnp.maximum(
        jnp.dot(_bf16r(pooled), _bf16r(hw[...]),
                preferred_element_type=jnp.float32) + hb[...],
        0.0)
    logits = jnp.dot(_bf16r(h), _bf16r(ow[...]),
                     preferred_element_type=jnp.float32) + ob[...]
    col = lax.broadcasted_iota(jnp.int32, (1, OUTPAD), 1)
    logits = jnp.where(col < 10, logits, -jnp.inf)
    m = jnp.max(logits, axis=1, keepdims=True)
    lse = jnp.log(jnp.sum(jnp.exp(logits - m), axis=1, keepdims=True)) + m
    out_ref[...] = logits - lse


_final_kernel = pl.pallas_call(
    _final_body,
    out_shape=jax.ShapeDtypeStruct((1, OUTPAD), jnp.float32),
)


@jax.jit
def kernel(node_features, edge_index, gcn_w, gcn_b, ln_w, ln_b,
           hid_w, hid_b, out_w, out_b):
    zeros = jnp.zeros((NPAD,), jnp.float32)
    edge_flat = edge_index.reshape(2 * E)

    deg_part, xpk = _deg_kernel(edge_flat, node_features, zeros)
    dinv = _scale_kernel(deg_part.reshape(NW * NPAD // 128, 128))
    acc_a, acc_b = _agg_kernel(edge_flat, xpk, dinv.reshape(NPAD), zeros)

    ob_pad = jnp.pad(out_b, (0, OUTPAD - 10)).reshape(1, OUTPAD)
    ow_pad = jnp.pad(out_w, ((0, 0), (0, OUTPAD - 10)))
    logits = _final_kernel(
        acc_a.reshape(NW * NPAD // 128, 128),
        acc_b.reshape(NW * NPAD // 128, 128),
        dinv,
        gcn_w, gcn_b.reshape(1, HID), ln_w.reshape(1, HID),
        ln_b.reshape(1, HID), hid_w, hid_b.reshape(1, HID),
        ow_pad, ob_pad,
    )
    return logits[:, :10]


# speedup vs baseline: 1.0741x; 1.0741x over previous
import functools

import jax
import jax.numpy as jnp
from jax import lax
from jax.experimental import pallas as pl
from jax.experimental.pallas import tpu as pltpu
from jax.experimental.pallas import tpu_sc as plsc

N = 10000
E = 320000
HID = 256
NC = 2
NS = 16
NW = NC * NS
NPAD = 10240
NPAD2 = 2 * NPAD
SLICE = NPAD // NS
EPT = E // NW
BLK = 2000
NBLK = EPT // BLK
UN = 5
OUTPAD = 128

_mesh = plsc.VectorSubcoreMesh(core_axis_name="c", subcore_axis_name="s")
_sc_params = pltpu.CompilerParams(needs_layout_passes=False)


def _merge(dst_ref, src_ref):
    def body(i, _):
        for u in range(UN):
            sl = pl.ds((i * UN + u) * 16, 16)
            dst_ref[sl] = dst_ref[sl] + src_ref[sl]
        return 0

    lax.fori_loop(0, NPAD // 16 // UN, body, 0)


@functools.partial(
    pl.kernel,
    out_type=jax.ShapeDtypeStruct((NW * NPAD,), jnp.float32),
    mesh=_mesh,
    compiler_params=_sc_params,
    scratch_types=[
        pltpu.VMEM((BLK,), jnp.int32),
        pltpu.VMEM((BLK,), jnp.int32),
        pltpu.VMEM((NPAD,), jnp.float32),
        pltpu.VMEM((NPAD,), jnp.float32),
        pltpu.SemaphoreType.DMA,
        pltpu.SemaphoreType.DMA,
    ],
)
def _deg_kernel(edge_hbm, zeros_hbm, deg_out,
                idx0, idx1, acc0, acc1, sem0, sem1):
    cid = lax.axis_index("c")
    sid = lax.axis_index("s")
    wid = cid * NS + sid
    base = wid * EPT
    bufs = (idx0, idx1)
    sems = (sem0, sem1)
    ones16 = jnp.ones((16,), jnp.float32)

    d0 = pltpu.async_copy(edge_hbm.at[pl.ds(E + base, BLK)], idx0, sem0)
    pltpu.sync_copy(zeros_hbm, acc0)
    pltpu.sync_copy(zeros_hbm, acc1)
    descs = [d0]
    accs = (acc0, acc1)

    for j in range(NBLK):
        if j + 1 < NBLK:
            descs.append(pltpu.async_copy(
                edge_hbm.at[pl.ds(E + base + (j + 1) * BLK, BLK)],
                bufs[(j + 1) % 2], sems[(j + 1) % 2]))
        descs[j].wait()
        buf = bufs[j % 2]

        def inner(i, _):
            for u in range(UN):
                d = buf[pl.ds((i * UN + u) * 16, 16)]
                plsc.addupdate_scatter(accs[u % 2], [d], ones16)
            return 0

        lax.fori_loop(0, BLK // 16 // UN, inner, 0)

    _merge(acc0, acc1)
    pltpu.sync_copy(acc0, deg_out.at[pl.ds(wid * NPAD, NPAD)])


def _scale_body(deg_parts, xflat, dinv_o, xr_o):
    deg = jnp.sum(deg_parts[...].reshape(NW, NPAD // 128, 128), axis=0) + 1.0
    dinv_o[...] = 1.0 / jnp.sqrt(deg)
    xr_o[...] = xflat[...].astype(jnp.bfloat16).astype(jnp.float32)


_scale_kernel = pl.pallas_call(
    _scale_body,
    out_shape=[
        jax.ShapeDtypeStruct((NPAD // 128, 128), jnp.float32),
        jax.ShapeDtypeStruct((NPAD2 // 128, 128), jnp.float32),
    ],
)


@functools.partial(
    pl.kernel,
    out_type=(
        jax.ShapeDtypeStruct((NW * NPAD,), jnp.float32),
        jax.ShapeDtypeStruct((NW * NPAD,), jnp.float32),
    ),
    mesh=_mesh,
    compiler_params=_sc_params,
    scratch_types=[
        pltpu.VMEM((NPAD2,), jnp.float32),
        pltpu.VMEM((NPAD,), jnp.float32),
        pltpu.VMEM((NPAD,), jnp.float32),
        pltpu.VMEM((NPAD,), jnp.float32),
        pltpu.VMEM((NPAD,), jnp.float32),
        pltpu.VMEM((NPAD,), jnp.float32),
        pltpu.VMEM((BLK,), jnp.int32),
        pltpu.VMEM((BLK,), jnp.int32),
        pltpu.VMEM((BLK,), jnp.int32),
        pltpu.VMEM((BLK,), jnp.int32),
        pltpu.SemaphoreType.DMA,
        pltpu.SemaphoreType.DMA,
    ],
)
def _agg_kernel(edge_hbm, xr_hbm, dinv_hbm, zeros_hbm, a_out, b_out,
                xr_v, dinv_v, acc_a0, acc_b0, acc_a1, acc_b1,
                src0, dst0, src1, dst1, sem0, sem1):
    cid = lax.axis_index("c")
    sid = lax.axis_index("s")
    wid = cid * NS + sid
    base = wid * EPT
    sbufs = (src0, src1)
    dbufs = (dst0, dst1)
    sems = (sem0, sem1)
    accs = ((acc_a0, acc_b0), (acc_a1, acc_b1))

    descs = [(pltpu.async_copy(edge_hbm.at[pl.ds(base, BLK)], src0, sem0),
              pltpu.async_copy(edge_hbm.at[pl.ds(E + base, BLK)], dst0, sem0))]
    pltpu.sync_copy(xr_hbm, xr_v)
    pltpu.sync_copy(dinv_hbm, dinv_v)
    pltpu.sync_copy(zeros_hbm, acc_a0)
    pltpu.sync_copy(zeros_hbm, acc_b0)
    pltpu.sync_copy(zeros_hbm, acc_a1)
    pltpu.sync_copy(zeros_hbm, acc_b1)

    iota16 = jax.lax.iota(jnp.int32, 16)

    @pl.when(cid == 0)
    def _selfloop():
        nbase = sid * SLICE

        def sbody(i, _):
            off = nbase + i * 16
            idx2 = (off + iota16) * 2
            va = plsc.load_gather(xr_v, [idx2])
            vb = plsc.load_gather(xr_v, [idx2 + 1])
            dv = dinv_v[pl.ds(off, 16)]
            acc_a0[pl.ds(off, 16)] = va * dv
            acc_b0[pl.ds(off, 16)] = vb * dv
            return 0

        lax.fori_loop(0, SLICE // 16, sbody, 0)

    for j in range(NBLK):
        if j + 1 < NBLK:
            nb = (j + 1) % 2
            descs.append((
                pltpu.async_copy(
                    edge_hbm.at[pl.ds(base + (j + 1) * BLK, BLK)],
                    sbufs[nb], sems[nb]),
                pltpu.async_copy(
                    edge_hbm.at[pl.ds(E + base + (j + 1) * BLK, BLK)],
                    dbufs[nb], sems[nb])))
        descs[j][0].wait()
        descs[j][1].wait()
        sbuf = sbufs[j % 2]
        dbuf = dbufs[j % 2]

        def inner(i, _):
            for u in range(UN):
                sl = pl.ds((i * UN + u) * 16, 16)
                s = sbuf[sl]
                d = dbuf[sl]
                s2 = s + s
                va = plsc.load_gather(xr_v, [s2])
                vb = plsc.load_gather(xr_v, [s2 + 1])
                dv = plsc.load_gather(dinv_v, [s])
                aa, bb = accs[u % 2]
                plsc.addupdate_scatter(aa, [d], va * dv)
                plsc.addupdate_scatter(bb, [d], vb * dv)
            return 0

        lax.fori_loop(0, BLK // 16 // UN, inner, 0)

    _merge(acc_a0, acc_a1)
    _merge(acc_b0, acc_b1)
    pltpu.sync_copy(acc_a0, a_out.at[pl.ds(wid * NPAD, NPAD)])
    pltpu.sync_copy(acc_b0, b_out.at[pl.ds(wid * NPAD, NPAD)])


def _final_body(acca, accb, dinv, gw, gb, lnw, lnb, hw, hb, ow, ob, out_ref):
    CH = 1024
    R = NPAD // 128

    A = acca[pl.ds(0, R), :]
    B = accb[pl.ds(0, R), :]
    for w in range(1, NW):
        A = A + acca[pl.ds(w * R, R), :]
        B = B + accb[pl.ds(w * R, R), :]
    DV = dinv[...]

    def _bf16r(v):
        return v.astype(jnp.bfloat16).astype(jnp.float32)

    gwr = _bf16r(gw[...])
    pooled = jnp.zeros((1, HID), jnp.float32)
    for i in range(NPAD // CH):
        aa = A[i * 8:(i + 1) * 8, :].reshape(CH)
        bb = B[i * 8:(i + 1) * 8, :].reshape(CH)
        dv = DV[i * 8:(i + 1) * 8, :].reshape(CH)
        g = ((aa * dv)[:, None] * gwr[0][None, :]
             + (bb * dv)[:, None] * gwr[1][None, :] + gb[...])
        g = jnp.maximum(g, 0.0)
        mean = jnp.mean(g, axis=1, keepdims=True)
        cen = g - mean
        var = jnp.mean(cen * cen, axis=1, keepdims=True)
        xln = cen / jnp.sqrt(var + 1e-5) * lnw[...] + lnb[...]
        rows = i * CH + lax.broadcasted_iota(jnp.int32, (CH, 1), 0)
        xln = jnp.where(rows < N, xln, 0.0)
        pooled = pooled + jnp.sum(xln, axis=0, keepdims=True)

    h = jnp.maximum(
        jnp.dot(_bf16r(pooled), _bf16r(hw[...]),
                preferred_element_type=jnp.float32) + hb[...],
        0.0)
    logits = jnp.dot(_bf16r(h), _bf16r(ow[...]),
                     preferred_element_type=jnp.float32) + ob[...]
    col = lax.broadcasted_iota(jnp.int32, (1, OUTPAD), 1)
    logits = jnp.where(col < 10, logits, -jnp.inf)
    m = jnp.max(logits, axis=1, keepdims=True)
    lse = jnp.log(jnp.sum(jnp.exp(logits - m), axis=1, keepdims=True)) + m
    out_ref[...] = logits - lse


_final_kernel = pl.pallas_call(
    _final_body,
    out_shape=jax.ShapeDtypeStruct((1, OUTPAD), jnp.float32),
)


@jax.jit
def kernel(node_features, edge_index, gcn_w, gcn_b, ln_w, ln_b,
           hid_w, hid_b, out_w, out_b):
    zeros = jnp.zeros((NPAD,), jnp.float32)
    edge_flat = edge_index.reshape(2 * E)
    xflat = jnp.pad(node_features.reshape(2 * N), (0, NPAD2 - 2 * N))

    deg_part = _deg_kernel(edge_flat, zeros)
    dinv, xr = _scale_kernel(
        deg_part.reshape(NW * NPAD // 128, 128),
        xflat.reshape(NPAD2 // 128, 128),
    )
    acc_a, acc_b = _agg_kernel(edge_flat, xr.reshape(NPAD2),
                               dinv.reshape(NPAD), zeros)

    ob_pad = jnp.pad(out_b, (0, OUTPAD - 10)).reshape(1, OUTPAD)
    ow_pad = jnp.pad(out_w, ((0, 0), (0, OUTPAD - 10)))
    logits = _final_kernel(
        acc_a.reshape(NW * NPAD // 128, 128),
        acc_b.reshape(NW * NPAD // 128, 128),
        dinv,
        gcn_w, gcn_b.reshape(1, HID), ln_w.reshape(1, HID),
        ln_b.reshape(1, HID), hid_w, hid_b.reshape(1, HID),
        ow_pad, ob_pad,
    )
    return logits[:, :10]
